# baseline (speedup 1.0000x reference)
import functools
import jax
import jax.numpy as jnp
from jax import lax
from jax.experimental import pallas as pl
from jax.experimental.pallas import tpu as pltpu
from jax.experimental.pallas import tpu_sc as plsc

N = 10000
E = 320000
F = 128
NP = 10240
NC, NS = 2, 16
NW = NC * NS
EPW = E // NW
K = 80
NCH = EPW // K
KC = 128
EPT = 10240
NCHC = EPT // KC
HCH = NCHC // 2
EPAD = NW * EPT - E
RPT = NP // NS

_mesh = functools.partial(
    plsc.VectorSubcoreMesh, core_axis_name="c", subcore_axis_name="s",
    num_cores=NC, num_subcores=NS)


@functools.partial(
    pl.kernel,
    out_type=jax.ShapeDtypeStruct((NC, NP), jnp.float32),
    mesh=_mesh(),
    scratch_types=[
        pltpu.VMEM((NCH, K), jnp.int32),
        pltpu.VMEM((K,), jnp.float32),
        pltpu.VMEM_SHARED((NP,), jnp.float32),
        pltpu.SemaphoreType.DMA,
    ],
)
def _sc_degree(rows_hbm, zeros_hbm, out_hbm, rowv, ones_v, hist, sem):
    c = lax.axis_index("c")
    s = lax.axis_index("s")
    pltpu.sync_copy(zeros_hbm.at[pl.ds(s * RPT, RPT)],
                    hist.at[pl.ds(s * RPT, RPT)])
    pltpu.sync_copy(rows_hbm.at[c, s], rowv)
    for i in range(K // 16):
        ones_v[pl.ds(i * 16, 16)] = jnp.ones((16,), jnp.float32)
    plsc.subcore_barrier()

    def body(j, carry):
        pltpu.sync_copy(ones_v, hist.at[rowv.at[j]], add=True)
        return carry

    lax.fori_loop(0, NCH, body, 0)
    plsc.subcore_barrier()
    pltpu.sync_copy(hist.at[pl.ds(s * RPT, RPT)],
                    out_hbm.at[c, pl.ds(s * RPT, RPT)])


@functools.partial(
    pl.kernel,
    out_type=jax.ShapeDtypeStruct((NC, NP, F), jnp.float32),
    mesh=_mesh(),
    scratch_types=[
        pltpu.VMEM((HCH, KC), jnp.int32),
        pltpu.VMEM((HCH, KC), jnp.int32),
        pltpu.VMEM((KC, F), jnp.float32),
        pltpu.VMEM((KC, F), jnp.float32),
        pltpu.VMEM_SHARED((NP, F), jnp.float32),
        pltpu.SemaphoreType.DMA,
        pltpu.SemaphoreType.DMA,
    ],
)
def _sc_aggregate(cols_hbm, rows_hbm, y_hbm, zeros_hbm, out_hbm,
                  colv, rowv, yb0, yb1, agg, gs0, gs1):
    c = lax.axis_index("c")
    s = lax.axis_index("s")
    pltpu.sync_copy(zeros_hbm, agg.at[pl.ds(s * RPT, RPT)])
    plsc.subcore_barrier()

    ybufs = (yb0, yb1)
    gsems = (gs0, gs1)

    for h in (0, 1):
        pltpu.sync_copy(cols_hbm.at[c, s, pl.ds(h * HCH, HCH)], colv)
        pltpu.sync_copy(rows_hbm.at[c, s, pl.ds(h * HCH, HCH)], rowv)
        pltpu.async_copy(y_hbm.at[colv.at[0]], ybufs[0], gsems[0])

        def body(t, carry):
            j = 2 * t
            for b in (0, 1):
                jj = j + b
                pltpu.make_async_copy(y_hbm.at[colv.at[jj]], ybufs[b],
                                      gsems[b]).wait()
                pltpu.async_copy(y_hbm.at[colv.at[jj + 1]], ybufs[1 - b],
                                 gsems[1 - b])
                pltpu.sync_copy(ybufs[b], agg.at[rowv.at[jj]], add=True)
            return carry

        lax.fori_loop(0, (HCH - 2) // 2, body, 0)
        for b in (0, 1):
            jj = HCH - 2 + b
            pltpu.make_async_copy(y_hbm.at[colv.at[jj]], ybufs[b],
                                  gsems[b]).wait()
            if jj + 1 < HCH:
                pltpu.async_copy(y_hbm.at[colv.at[jj + 1]], ybufs[1 - b],
                                 gsems[1 - b])
            pltpu.sync_copy(ybufs[b], agg.at[rowv.at[jj]], add=True)
    plsc.subcore_barrier()
    pltpu.sync_copy(agg.at[pl.ds(s * RPT, RPT)],
                    out_hbm.at[c, pl.ds(s * RPT, RPT)])


def _tc_scale_body(h0, h1, x, y):
    d = h0[...] + h1[...] + 1.0
    dinv = lax.rsqrt(d)
    y[...] = x[...] * dinv


BN = 1024

_tc_scale = pl.pallas_call(
    _tc_scale_body,
    out_shape=jax.ShapeDtypeStruct((NP, F), jnp.float32),
    grid=(NP // BN,),
    in_specs=[
        pl.BlockSpec((BN, 1), lambda i: (i, 0)),
        pl.BlockSpec((BN, 1), lambda i: (i, 0)),
        pl.BlockSpec((BN, F), lambda i: (i, 0)),
    ],
    out_specs=pl.BlockSpec((BN, F), lambda i: (i, 0)),
)


def _tc_final_body(h0, h1, y, a0, a1, w, bias, out):
    d = h0[...] + h1[...] + 1.0
    dinv = lax.rsqrt(d)
    sagg = (a0[...] + a1[...] + y[...]) * dinv
    z = jnp.dot(sagg, w[...], preferred_element_type=jnp.float32) + bias[...]
    zn = jnp.minimum(z, 0.0)
    out[...] = jnp.where(z > 0, z, jnp.exp(zn) - 1.0)


_tc_final = pl.pallas_call(
    _tc_final_body,
    out_shape=jax.ShapeDtypeStruct((NP, F), jnp.float32),
    grid=(NP // BN,),
    in_specs=[
        pl.BlockSpec((BN, 1), lambda i: (i, 0)),
        pl.BlockSpec((BN, 1), lambda i: (i, 0)),
        pl.BlockSpec((BN, F), lambda i: (i, 0)),
        pl.BlockSpec((BN, F), lambda i: (i, 0)),
        pl.BlockSpec((BN, F), lambda i: (i, 0)),
        pl.BlockSpec((F, F), lambda i: (0, 0)),
        pl.BlockSpec((1, F), lambda i: (0, 0)),
    ],
    out_specs=pl.BlockSpec((BN, F), lambda i: (i, 0)),
)


@jax.jit
def kernel(x, edge_index, W, b):
    xp = jnp.pad(x.reshape(N, F), ((0, NP - N), (0, 0)))
    rows_r = edge_index[0].reshape(NC, NS, NCH, K)
    cols_r = edge_index[1].reshape(NC, NS, NCH, K)
    zrow = jnp.zeros((NP,), jnp.float32)
    zagg = jnp.zeros((RPT, F), jnp.float32)

    epad = jnp.full((EPAD,), NP - 1, jnp.int32)
    rows_c = jnp.concatenate([edge_index[0], epad]).reshape(NC, NS, NCHC, KC)
    cols_c = jnp.concatenate([edge_index[1], epad]).reshape(NC, NS, NCHC, KC)

    hist2 = _sc_degree(rows_r, zrow)
    h0 = hist2[0].reshape(NP, 1)
    h1 = hist2[1].reshape(NP, 1)
    y = _tc_scale(h0, h1, xp)
    agg2 = _sc_aggregate(cols_c, rows_c, y, zagg)
    out = _tc_final(h0, h1, y, agg2[0], agg2[1], W, b.reshape(1, F))
    return out[:N].reshape(1, N, F)


# speedup vs baseline: 2.7986x; 2.7986x over previous
import functools
import jax
import jax.numpy as jnp
from jax import lax
from jax.experimental import pallas as pl
from jax.experimental.pallas import tpu as pltpu
from jax.experimental.pallas import tpu_sc as plsc

N = 10000
E = 320000
F = 128
NP = 10240
NC, NS = 2, 16
NW = NC * NS
EPW = E // NW
K = 80
NCH = EPW // K
KC = 128
EPT = 10240
NCHC = EPT // KC
HCH = NCHC // 2
EPAD = NW * EPT - E
RPT = NP // NS

_mesh = functools.partial(
    plsc.VectorSubcoreMesh, core_axis_name="c", subcore_axis_name="s",
    num_cores=NC, num_subcores=NS)


@functools.partial(
    pl.kernel,
    out_type=jax.ShapeDtypeStruct((NC, NP), jnp.float32),
    mesh=_mesh(),
    scratch_types=[
        pltpu.VMEM((NCH, K), jnp.int32),
        pltpu.VMEM((K,), jnp.float32),
        pltpu.VMEM_SHARED((NP,), jnp.float32),
        pltpu.SemaphoreType.DMA,
    ],
)
def _sc_degree(rows_hbm, zeros_hbm, out_hbm, rowv, ones_v, hist, sem):
    c = lax.axis_index("c")
    s = lax.axis_index("s")
    pltpu.sync_copy(zeros_hbm.at[pl.ds(s * RPT, RPT)],
                    hist.at[pl.ds(s * RPT, RPT)])
    pltpu.sync_copy(rows_hbm.at[c, s], rowv)
    for i in range(K // 16):
        ones_v[pl.ds(i * 16, 16)] = jnp.ones((16,), jnp.float32)
    plsc.subcore_barrier()

    def body(j, carry):
        pltpu.sync_copy(ones_v, hist.at[rowv.at[j]], add=True)
        return carry

    lax.fori_loop(0, NCH, body, 0)
    plsc.subcore_barrier()
    pltpu.sync_copy(hist.at[pl.ds(s * RPT, RPT)],
                    out_hbm.at[c, pl.ds(s * RPT, RPT)])


@functools.partial(
    pl.kernel,
    out_type=jax.ShapeDtypeStruct((NC, NP, F), jnp.float32),
    mesh=_mesh(),
    scratch_types=[
        pltpu.VMEM((HCH, KC), jnp.int32),
        pltpu.VMEM((HCH, KC), jnp.int32),
        pltpu.VMEM((KC, F), jnp.float32),
        pltpu.VMEM((KC, F), jnp.float32),
        pltpu.VMEM_SHARED((NP, F), jnp.float32),
        pltpu.SemaphoreType.DMA,
        pltpu.SemaphoreType.DMA,
    ],
)
def _sc_aggregate(cols_hbm, rows_hbm, y_hbm, zeros_hbm, out_hbm,
                  colv, rowv, yb0, yb1, agg, gs0, gs1):
    c = lax.axis_index("c")
    s = lax.axis_index("s")
    pltpu.sync_copy(zeros_hbm, agg.at[pl.ds(s * RPT, RPT)])
    plsc.subcore_barrier()

    ybufs = (yb0, yb1)
    gsems = (gs0, gs1)

    for h in (0, 1):
        pltpu.sync_copy(cols_hbm.at[c, s, pl.ds(h * HCH, HCH)], colv)
        pltpu.sync_copy(rows_hbm.at[c, s, pl.ds(h * HCH, HCH)], rowv)
        pltpu.async_copy(y_hbm.at[colv.at[0]], ybufs[0], gsems[0])

        def body(t, carry):
            j = 2 * t
            for b in (0, 1):
                jj = j + b
                pltpu.make_async_copy(y_hbm.at[colv.at[jj]], ybufs[b],
                                      gsems[b]).wait()
                pltpu.async_copy(y_hbm.at[colv.at[jj + 1]], ybufs[1 - b],
                                 gsems[1 - b])
                pltpu.sync_copy(ybufs[b], agg.at[rowv.at[jj]], add=True)
            return carry

        lax.fori_loop(0, (HCH - 2) // 2, body, 0)
        for b in (0, 1):
            jj = HCH - 2 + b
            pltpu.make_async_copy(y_hbm.at[colv.at[jj]], ybufs[b],
                                  gsems[b]).wait()
            if jj + 1 < HCH:
                pltpu.async_copy(y_hbm.at[colv.at[jj + 1]], ybufs[1 - b],
                                 gsems[1 - b])
            pltpu.sync_copy(ybufs[b], agg.at[rowv.at[jj]], add=True)
    plsc.subcore_barrier()
    pltpu.sync_copy(agg.at[pl.ds(s * RPT, RPT)],
                    out_hbm.at[c, pl.ds(s * RPT, RPT)])


def _tc_scale_body(h0, h1, x, y):
    d = h0[...] + h1[...] + 1.0
    dinv = lax.rsqrt(d)
    y[...] = x[...] * dinv


BN = 1024

_tc_scale = pl.pallas_call(
    _tc_scale_body,
    out_shape=jax.ShapeDtypeStruct((NP, F), jnp.float32),
    grid=(NP // BN,),
    in_specs=[
        pl.BlockSpec((BN, 1), lambda i: (i, 0)),
        pl.BlockSpec((BN, 1), lambda i: (i, 0)),
        pl.BlockSpec((BN, F), lambda i: (i, 0)),
    ],
    out_specs=pl.BlockSpec((BN, F), lambda i: (i, 0)),
)


def _tc_final_body(h0, h1, y, a0, a1, w, bias, out):
    d = h0[...] + h1[...] + 1.0
    dinv = lax.rsqrt(d)
    sagg = (a0[...] + a1[...] + y[...]) * dinv
    z = jnp.dot(sagg, w[...], preferred_element_type=jnp.float32) + bias[...]
    zn = jnp.minimum(z, 0.0)
    out[...] = jnp.where(z > 0, z, jnp.exp(zn) - 1.0)


_tc_final = pl.pallas_call(
    _tc_final_body,
    out_shape=jax.ShapeDtypeStruct((NP, F), jnp.float32),
    grid=(NP // BN,),
    in_specs=[
        pl.BlockSpec((BN, 1), lambda i: (i, 0)),
        pl.BlockSpec((BN, 1), lambda i: (i, 0)),
        pl.BlockSpec((BN, F), lambda i: (i, 0)),
        pl.BlockSpec((BN, F), lambda i: (i, 0)),
        pl.BlockSpec((BN, F), lambda i: (i, 0)),
        pl.BlockSpec((F, F), lambda i: (0, 0)),
        pl.BlockSpec((1, F), lambda i: (0, 0)),
    ],
    out_specs=pl.BlockSpec((BN, F), lambda i: (i, 0)),
)


@jax.jit
def kernel(x, edge_index, W, b):
    xp = jnp.pad(x.reshape(N, F), ((0, NP - N), (0, 0)))
    rows_r = edge_index[0].reshape(NC, NS, NCH, K)
    cols_r = edge_index[1].reshape(NC, NS, NCH, K)
    zrow = jnp.zeros((NP,), jnp.float32)
    zagg = jnp.zeros((RPT, F), jnp.float32)

    epad = (N + jnp.arange(EPAD, dtype=jnp.int32) % (NP - N)).astype(jnp.int32)
    rows_c = jnp.concatenate([edge_index[0], epad]).reshape(NC, NS, NCHC, KC)
    cols_c = jnp.concatenate([edge_index[1], epad]).reshape(NC, NS, NCHC, KC)

    hist2 = _sc_degree(rows_r, zrow)
    h0 = hist2[0].reshape(NP, 1)
    h1 = hist2[1].reshape(NP, 1)
    y = _tc_scale(h0, h1, xp)
    agg2 = _sc_aggregate(cols_c, rows_c, y, zagg)
    out = _tc_final(h0, h1, y, agg2[0], agg2[1], W, b.reshape(1, F))
    return out[:N].reshape(1, N, F)


# speedup vs baseline: 2.8001x; 1.0005x over previous
import functools
import jax
import jax.numpy as jnp
from jax import lax
from jax.experimental import pallas as pl
from jax.experimental.pallas import tpu as pltpu
from jax.experimental.pallas import tpu_sc as plsc

N = 10000
E = 320000
F = 128
NP = 10240
NC, NS = 2, 16
NW = NC * NS
EPW = E // NW
K = 80
NCH = EPW // K
KC = 128
EPT = 10240
NCHC = EPT // KC
HCH = NCHC // 2
EPAD = NW * EPT - E
RPT = NP // NS

_mesh = functools.partial(
    plsc.VectorSubcoreMesh, core_axis_name="c", subcore_axis_name="s",
    num_cores=NC, num_subcores=NS)


@functools.partial(
    pl.kernel,
    out_type=jax.ShapeDtypeStruct((NC, NP), jnp.float32),
    mesh=_mesh(),
    scratch_types=[
        pltpu.VMEM((NCH, K), jnp.int32),
        pltpu.VMEM((K,), jnp.float32),
        pltpu.VMEM_SHARED((NP,), jnp.float32),
        pltpu.SemaphoreType.DMA,
    ],
)
def _sc_degree(rows_hbm, zeros_hbm, out_hbm, rowv, ones_v, hist, sem):
    c = lax.axis_index("c")
    s = lax.axis_index("s")
    pltpu.sync_copy(zeros_hbm.at[pl.ds(s * RPT, RPT)],
                    hist.at[pl.ds(s * RPT, RPT)])
    pltpu.sync_copy(rows_hbm.at[c, s], rowv)
    for i in range(K // 16):
        ones_v[pl.ds(i * 16, 16)] = jnp.ones((16,), jnp.float32)
    plsc.subcore_barrier()

    def body(j, carry):
        pltpu.sync_copy(ones_v, hist.at[rowv.at[j]], add=True)
        return carry

    lax.fori_loop(0, NCH, body, 0)
    plsc.subcore_barrier()
    pltpu.sync_copy(hist.at[pl.ds(s * RPT, RPT)],
                    out_hbm.at[c, pl.ds(s * RPT, RPT)])


@functools.partial(
    pl.kernel,
    out_type=jax.ShapeDtypeStruct((NC, NP, F), jnp.float32),
    mesh=_mesh(),
    scratch_types=[
        pltpu.VMEM((HCH, KC), jnp.int32),
        pltpu.VMEM((HCH, KC), jnp.int32),
        pltpu.VMEM((KC, F), jnp.float32),
        pltpu.VMEM((KC, F), jnp.float32),
        pltpu.VMEM_SHARED((NP, F), jnp.float32),
        pltpu.SemaphoreType.DMA,
        pltpu.SemaphoreType.DMA,
        pltpu.SemaphoreType.DMA,
        pltpu.SemaphoreType.DMA,
    ],
)
def _sc_aggregate(cols_hbm, rows_hbm, y_hbm, zeros_hbm, out_hbm,
                  colv, rowv, yb0, yb1, agg, gs0, gs1, ss0, ss1):
    c = lax.axis_index("c")
    s = lax.axis_index("s")
    pltpu.sync_copy(zeros_hbm, agg.at[pl.ds(s * RPT, RPT)])
    plsc.subcore_barrier()

    ybufs = (yb0, yb1)
    gsems = (gs0, gs1)
    ssems = (ss0, ss1)

    for h in (0, 1):
        pltpu.sync_copy(cols_hbm.at[c, s, pl.ds(h * HCH, HCH)], colv)
        pltpu.sync_copy(rows_hbm.at[c, s, pl.ds(h * HCH, HCH)], rowv)
        pltpu.async_copy(y_hbm.at[colv.at[0]], ybufs[0], gsems[0])

        pltpu.make_async_copy(y_hbm.at[colv.at[0]], ybufs[0], gsems[0]).wait()
        pltpu.async_copy(ybufs[0], agg.at[rowv.at[0]], ssems[0], add=True)
        pltpu.async_copy(y_hbm.at[colv.at[1]], ybufs[1], gsems[1])

        def body(t, carry):
            j = 2 * t + 1
            for b in (1, 0):
                jj = j + (1 - b)
                pltpu.make_async_copy(y_hbm.at[colv.at[jj]], ybufs[b],
                                      gsems[b]).wait()
                pltpu.async_copy(ybufs[b], agg.at[rowv.at[jj]], ssems[b],
                                 add=True)
                pltpu.make_async_copy(ybufs[1 - b], agg.at[rowv.at[jj]],
                                      ssems[1 - b]).wait()
                pltpu.async_copy(y_hbm.at[colv.at[jj + 1]], ybufs[1 - b],
                                 gsems[1 - b])
            return carry

        lax.fori_loop(0, (HCH - 2) // 2, body, 0)
        pltpu.make_async_copy(y_hbm.at[colv.at[HCH - 1]], ybufs[1],
                              gsems[1]).wait()
        pltpu.async_copy(ybufs[1], agg.at[rowv.at[HCH - 1]], ssems[1],
                         add=True)
        pltpu.make_async_copy(ybufs[0], agg.at[rowv.at[HCH - 2]],
                              ssems[0]).wait()
        pltpu.make_async_copy(ybufs[1], agg.at[rowv.at[HCH - 1]],
                              ssems[1]).wait()
    plsc.subcore_barrier()
    pltpu.sync_copy(agg.at[pl.ds(s * RPT, RPT)],
                    out_hbm.at[c, pl.ds(s * RPT, RPT)])


def _tc_scale_body(h0, h1, x, y):
    d = h0[...] + h1[...] + 1.0
    dinv = lax.rsqrt(d)
    y[...] = x[...] * dinv


BN = 1024

_tc_scale = pl.pallas_call(
    _tc_scale_body,
    out_shape=jax.ShapeDtypeStruct((NP, F), jnp.float32),
    grid=(NP // BN,),
    in_specs=[
        pl.BlockSpec((BN, 1), lambda i: (i, 0)),
        pl.BlockSpec((BN, 1), lambda i: (i, 0)),
        pl.BlockSpec((BN, F), lambda i: (i, 0)),
    ],
    out_specs=pl.BlockSpec((BN, F), lambda i: (i, 0)),
)


def _tc_final_body(h0, h1, y, a0, a1, w, bias, out):
    d = h0[...] + h1[...] + 1.0
    dinv = lax.rsqrt(d)
    sagg = (a0[...] + a1[...] + y[...]) * dinv
    z = jnp.dot(sagg, w[...], preferred_element_type=jnp.float32) + bias[...]
    zn = jnp.minimum(z, 0.0)
    out[...] = jnp.where(z > 0, z, jnp.exp(zn) - 1.0)


_tc_final = pl.pallas_call(
    _tc_final_body,
    out_shape=jax.ShapeDtypeStruct((NP, F), jnp.float32),
    grid=(NP // BN,),
    in_specs=[
        pl.BlockSpec((BN, 1), lambda i: (i, 0)),
        pl.BlockSpec((BN, 1), lambda i: (i, 0)),
        pl.BlockSpec((BN, F), lambda i: (i, 0)),
        pl.BlockSpec((BN, F), lambda i: (i, 0)),
        pl.BlockSpec((BN, F), lambda i: (i, 0)),
        pl.BlockSpec((F, F), lambda i: (0, 0)),
        pl.BlockSpec((1, F), lambda i: (0, 0)),
    ],
    out_specs=pl.BlockSpec((BN, F), lambda i: (i, 0)),
)


@jax.jit
def kernel(x, edge_index, W, b):
    xp = jnp.pad(x.reshape(N, F), ((0, NP - N), (0, 0)))
    rows_r = edge_index[0].reshape(NC, NS, NCH, K)
    cols_r = edge_index[1].reshape(NC, NS, NCH, K)
    zrow = jnp.zeros((NP,), jnp.float32)
    zagg = jnp.zeros((RPT, F), jnp.float32)

    epad = (N + jnp.arange(EPAD, dtype=jnp.int32) % (NP - N)).astype(jnp.int32)
    rows_c = jnp.concatenate([edge_index[0], epad]).reshape(NC, NS, NCHC, KC)
    cols_c = jnp.concatenate([edge_index[1], epad]).reshape(NC, NS, NCHC, KC)

    hist2 = _sc_degree(rows_r, zrow)
    h0 = hist2[0].reshape(NP, 1)
    h1 = hist2[1].reshape(NP, 1)
    y = _tc_scale(h0, h1, xp)
    agg2 = _sc_aggregate(cols_c, rows_c, y, zagg)
    out = _tc_final(h0, h1, y, agg2[0], agg2[1], W, b.reshape(1, F))
    return out[:N].reshape(1, N, F)


# speedup vs baseline: 2.8944x; 1.0337x over previous
import functools
import jax
import jax.numpy as jnp
from jax import lax
from jax.experimental import pallas as pl
from jax.experimental.pallas import tpu as pltpu
from jax.experimental.pallas import tpu_sc as plsc

N = 10000
E = 320000
F = 128
NP = 10240
NC, NS = 2, 16
NW = NC * NS
EPW = E // NW
K = 80
NCH = EPW // K
KC = 125
NCHC = EPW // KC
HCH = NCHC // 2
RPT = NP // NS

_mesh = functools.partial(
    plsc.VectorSubcoreMesh, core_axis_name="c", subcore_axis_name="s",
    num_cores=NC, num_subcores=NS)


@functools.partial(
    pl.kernel,
    out_type=jax.ShapeDtypeStruct((NC, NP), jnp.float32),
    mesh=_mesh(),
    scratch_types=[
        pltpu.VMEM((NCH, K), jnp.int32),
        pltpu.VMEM((K,), jnp.float32),
        pltpu.VMEM_SHARED((NP,), jnp.float32),
        pltpu.SemaphoreType.DMA,
    ],
)
def _sc_degree(rows_hbm, zeros_hbm, out_hbm, rowv, ones_v, hist, sem):
    c = lax.axis_index("c")
    s = lax.axis_index("s")
    pltpu.sync_copy(zeros_hbm.at[pl.ds(s * RPT, RPT)],
                    hist.at[pl.ds(s * RPT, RPT)])
    pltpu.sync_copy(rows_hbm.at[c, s], rowv)
    for i in range(K // 16):
        ones_v[pl.ds(i * 16, 16)] = jnp.ones((16,), jnp.float32)
    plsc.subcore_barrier()

    def body(j, carry):
        pltpu.sync_copy(ones_v, hist.at[rowv.at[j]], add=True)
        return carry

    lax.fori_loop(0, NCH, body, 0)
    plsc.subcore_barrier()
    pltpu.sync_copy(hist.at[pl.ds(s * RPT, RPT)],
                    out_hbm.at[c, pl.ds(s * RPT, RPT)])


@functools.partial(
    pl.kernel,
    out_type=jax.ShapeDtypeStruct((NC, NP, F), jnp.float32),
    mesh=_mesh(),
    scratch_types=[
        pltpu.VMEM((HCH, KC), jnp.int32),
        pltpu.VMEM((HCH, KC), jnp.int32),
        pltpu.VMEM((KC, F), jnp.float32),
        pltpu.VMEM((KC, F), jnp.float32),
        pltpu.VMEM_SHARED((NP, F), jnp.float32),
        pltpu.SemaphoreType.DMA,
        pltpu.SemaphoreType.DMA,
        pltpu.SemaphoreType.DMA,
        pltpu.SemaphoreType.DMA,
    ],
)
def _sc_aggregate(cols_hbm, rows_hbm, y_hbm, zeros_hbm, out_hbm,
                  colv, rowv, yb0, yb1, agg, gs0, gs1, ss0, ss1):
    c = lax.axis_index("c")
    s = lax.axis_index("s")
    pltpu.sync_copy(zeros_hbm, agg.at[pl.ds(s * RPT, RPT)])
    plsc.subcore_barrier()

    ybufs = (yb0, yb1)
    gsems = (gs0, gs1)
    ssems = (ss0, ss1)

    for h in (0, 1):
        pltpu.sync_copy(cols_hbm.at[c, s, pl.ds(h * HCH, HCH)], colv)
        pltpu.sync_copy(rows_hbm.at[c, s, pl.ds(h * HCH, HCH)], rowv)
        pltpu.async_copy(y_hbm.at[colv.at[0]], ybufs[0], gsems[0])

        pltpu.make_async_copy(y_hbm.at[colv.at[0]], ybufs[0], gsems[0]).wait()
        pltpu.async_copy(ybufs[0], agg.at[rowv.at[0]], ssems[0], add=True)
        pltpu.async_copy(y_hbm.at[colv.at[1]], ybufs[1], gsems[1])

        def body(t, carry):
            j = 2 * t + 1
            for b in (1, 0):
                jj = j + (1 - b)
                pltpu.make_async_copy(y_hbm.at[colv.at[jj]], ybufs[b],
                                      gsems[b]).wait()
                pltpu.async_copy(ybufs[b], agg.at[rowv.at[jj]], ssems[b],
                                 add=True)
                pltpu.make_async_copy(ybufs[1 - b], agg.at[rowv.at[jj]],
                                      ssems[1 - b]).wait()
                pltpu.async_copy(y_hbm.at[colv.at[jj + 1]], ybufs[1 - b],
                                 gsems[1 - b])
            return carry

        lax.fori_loop(0, (HCH - 2) // 2, body, 0)
        pltpu.make_async_copy(y_hbm.at[colv.at[HCH - 1]], ybufs[1],
                              gsems[1]).wait()
        pltpu.async_copy(ybufs[1], agg.at[rowv.at[HCH - 1]], ssems[1],
                         add=True)
        pltpu.make_async_copy(ybufs[0], agg.at[rowv.at[HCH - 2]],
                              ssems[0]).wait()
        pltpu.make_async_copy(ybufs[1], agg.at[rowv.at[HCH - 1]],
                              ssems[1]).wait()
    plsc.subcore_barrier()
    pltpu.sync_copy(agg.at[pl.ds(s * RPT, RPT)],
                    out_hbm.at[c, pl.ds(s * RPT, RPT)])


def _tc_scale_body(h0, h1, x, y):
    d = h0[...] + h1[...] + 1.0
    dinv = lax.rsqrt(d)
    y[...] = x[...] * dinv


BN = 1000

_tc_scale = pl.pallas_call(
    _tc_scale_body,
    out_shape=jax.ShapeDtypeStruct((N, F), jnp.float32),
    grid=(N // BN,),
    in_specs=[
        pl.BlockSpec((BN, 1), lambda i: (i, 0)),
        pl.BlockSpec((BN, 1), lambda i: (i, 0)),
        pl.BlockSpec((BN, F), lambda i: (i, 0)),
    ],
    out_specs=pl.BlockSpec((BN, F), lambda i: (i, 0)),
)


def _tc_final_body(h0, h1, y, a0, a1, w, bias, out):
    d = h0[...] + h1[...] + 1.0
    dinv = lax.rsqrt(d)
    sagg = (a0[...] + a1[...] + y[...]) * dinv
    z = jnp.dot(sagg, w[...], preferred_element_type=jnp.float32) + bias[...]
    zn = jnp.minimum(z, 0.0)
    out[...] = jnp.where(z > 0, z, jnp.exp(zn) - 1.0)


_tc_final = pl.pallas_call(
    _tc_final_body,
    out_shape=jax.ShapeDtypeStruct((N, F), jnp.float32),
    grid=(N // BN,),
    in_specs=[
        pl.BlockSpec((BN, 1), lambda i: (i, 0)),
        pl.BlockSpec((BN, 1), lambda i: (i, 0)),
        pl.BlockSpec((BN, F), lambda i: (i, 0)),
        pl.BlockSpec((BN, F), lambda i: (i, 0)),
        pl.BlockSpec((BN, F), lambda i: (i, 0)),
        pl.BlockSpec((F, F), lambda i: (0, 0)),
        pl.BlockSpec((1, F), lambda i: (0, 0)),
    ],
    out_specs=pl.BlockSpec((BN, F), lambda i: (i, 0)),
)


@jax.jit
def kernel(x, edge_index, W, b):
    x2 = x.reshape(N, F)
    rows_r = edge_index[0].reshape(NC, NS, NCH, K)
    cols_r = edge_index[1].reshape(NC, NS, NCH, K)
    rows_c = edge_index[0].reshape(NC, NS, NCHC, KC)
    cols_c = edge_index[1].reshape(NC, NS, NCHC, KC)
    zrow = jnp.zeros((NP,), jnp.float32)
    zagg = jnp.zeros((RPT, F), jnp.float32)

    hist2 = _sc_degree(rows_r, zrow)
    h0 = hist2[0].reshape(NP, 1)
    h1 = hist2[1].reshape(NP, 1)
    y = _tc_scale(h0, h1, x2)
    agg2 = _sc_aggregate(cols_c, rows_c, y, zagg)
    out = _tc_final(h0, h1, y, agg2[0], agg2[1], W, b.reshape(1, F))
    return out.reshape(1, N, F)


# speedup vs baseline: 3.0175x; 1.0425x over previous
import functools
import jax
import jax.numpy as jnp
from jax import lax
from jax.experimental import pallas as pl
from jax.experimental.pallas import tpu as pltpu
from jax.experimental.pallas import tpu_sc as plsc

N = 10000
E = 320000
F = 128
NP = 10240
NC, NS = 2, 16
NW = NC * NS
EPW = E // NW
K = 80
NCH = EPW // K
KC = 125
NCHC = EPW // KC
HCH = NCHC // 2
RPT = NP // NS

_mesh = functools.partial(
    plsc.VectorSubcoreMesh, core_axis_name="c", subcore_axis_name="s",
    num_cores=NC, num_subcores=NS)


@functools.partial(
    pl.kernel,
    out_type=jax.ShapeDtypeStruct((NC, NP), jnp.float32),
    mesh=_mesh(),
    scratch_types=[
        pltpu.VMEM((NCH, K), jnp.int32),
        pltpu.VMEM((K,), jnp.float32),
        pltpu.VMEM_SHARED((NP,), jnp.float32),
        pltpu.SemaphoreType.DMA,
    ],
)
def _sc_degree(rows_hbm, zeros_hbm, out_hbm, rowv, ones_v, hist, sem):
    c = lax.axis_index("c")
    s = lax.axis_index("s")
    pltpu.sync_copy(zeros_hbm.at[pl.ds(s * RPT, RPT)],
                    hist.at[pl.ds(s * RPT, RPT)])
    pltpu.sync_copy(rows_hbm.at[c, s], rowv)
    for i in range(K // 16):
        ones_v[pl.ds(i * 16, 16)] = jnp.ones((16,), jnp.float32)
    plsc.subcore_barrier()

    def body(j, carry):
        pltpu.sync_copy(ones_v, hist.at[rowv.at[j]], add=True)
        return carry

    lax.fori_loop(0, NCH, body, 0)
    plsc.subcore_barrier()
    pltpu.sync_copy(hist.at[pl.ds(s * RPT, RPT)],
                    out_hbm.at[c, pl.ds(s * RPT, RPT)])


@functools.partial(
    pl.kernel,
    out_type=jax.ShapeDtypeStruct((NC, NP, F), jnp.float32),
    mesh=_mesh(),
    scratch_types=[
        pltpu.VMEM((HCH, KC), jnp.int32),
        pltpu.VMEM((HCH, KC), jnp.int32),
        pltpu.VMEM((KC, F), jnp.float32),
        pltpu.VMEM((KC, F), jnp.float32),
        pltpu.VMEM_SHARED((NP, F), jnp.float32),
        pltpu.SemaphoreType.DMA,
        pltpu.SemaphoreType.DMA,
        pltpu.SemaphoreType.DMA,
        pltpu.SemaphoreType.DMA,
    ],
)
def _sc_aggregate(cols_hbm, rows_hbm, y_hbm, zeros_hbm, out_hbm,
                  colv, rowv, yb0, yb1, agg, gs0, gs1, ss0, ss1):
    c = lax.axis_index("c")
    s = lax.axis_index("s")
    pltpu.sync_copy(zeros_hbm, agg.at[pl.ds(s * RPT, RPT)])
    plsc.subcore_barrier()

    ybufs = (yb0, yb1)
    gsems = (gs0, gs1)
    ssems = (ss0, ss1)

    for h in (0, 1):
        pltpu.sync_copy(cols_hbm.at[c, s, pl.ds(h * HCH, HCH)], colv)
        pltpu.sync_copy(rows_hbm.at[c, s, pl.ds(h * HCH, HCH)], rowv)
        pltpu.async_copy(y_hbm.at[colv.at[0]], ybufs[0], gsems[0])

        pltpu.make_async_copy(y_hbm.at[colv.at[0]], ybufs[0], gsems[0]).wait()
        pltpu.async_copy(ybufs[0], agg.at[rowv.at[0]], ssems[0], add=True)
        pltpu.async_copy(y_hbm.at[colv.at[1]], ybufs[1], gsems[1])

        def body(t, carry):
            j = 2 * t + 1
            for b in (1, 0):
                jj = j + (1 - b)
                pltpu.make_async_copy(y_hbm.at[colv.at[jj]], ybufs[b],
                                      gsems[b]).wait()
                pltpu.async_copy(ybufs[b], agg.at[rowv.at[jj]], ssems[b],
                                 add=True)
                pltpu.make_async_copy(ybufs[1 - b], agg.at[rowv.at[jj]],
                                      ssems[1 - b]).wait()
                pltpu.async_copy(y_hbm.at[colv.at[jj + 1]], ybufs[1 - b],
                                 gsems[1 - b])
            return carry

        lax.fori_loop(0, (HCH - 2) // 2, body, 0)
        pltpu.make_async_copy(y_hbm.at[colv.at[HCH - 1]], ybufs[1],
                              gsems[1]).wait()
        pltpu.async_copy(ybufs[1], agg.at[rowv.at[HCH - 1]], ssems[1],
                         add=True)
        pltpu.make_async_copy(ybufs[0], agg.at[rowv.at[HCH - 2]],
                              ssems[0]).wait()
        pltpu.make_async_copy(ybufs[1], agg.at[rowv.at[HCH - 1]],
                              ssems[1]).wait()
    plsc.subcore_barrier()
    pltpu.sync_copy(agg.at[pl.ds(s * RPT, RPT)],
                    out_hbm.at[c, pl.ds(s * RPT, RPT)])


def _tc_scale_body(h0, h1, x, y):
    d = h0[...] + h1[...] + 1.0
    dinv = lax.rsqrt(d)
    y[...] = x[...] * dinv.reshape(BN, 1)


BN = 1000

_tc_scale = pl.pallas_call(
    _tc_scale_body,
    out_shape=jax.ShapeDtypeStruct((N, F), jnp.float32),
    grid=(N // BN,),
    in_specs=[
        pl.BlockSpec((1, BN, 1), lambda i: (0, i, 0)),
        pl.BlockSpec((1, BN, 1), lambda i: (1, i, 0)),
        pl.BlockSpec((BN, F), lambda i: (i, 0)),
    ],
    out_specs=pl.BlockSpec((BN, F), lambda i: (i, 0)),
)


def _tc_final_body(h0, h1, y, a0, a1, w, bias, out):
    d = h0[...] + h1[...] + 1.0
    dinv = lax.rsqrt(d).reshape(BN, 1)
    sagg = (a0[...].reshape(BN, F) + a1[...].reshape(BN, F) + y[...]) * dinv
    z = jnp.dot(sagg, w[...], preferred_element_type=jnp.float32) + bias[...]
    zn = jnp.minimum(z, 0.0)
    out[...] = jnp.where(z > 0, z, jnp.exp(zn) - 1.0)


_tc_final = pl.pallas_call(
    _tc_final_body,
    out_shape=jax.ShapeDtypeStruct((N, F), jnp.float32),
    grid=(N // BN,),
    in_specs=[
        pl.BlockSpec((1, BN, 1), lambda i: (0, i, 0)),
        pl.BlockSpec((1, BN, 1), lambda i: (1, i, 0)),
        pl.BlockSpec((BN, F), lambda i: (i, 0)),
        pl.BlockSpec((1, BN, F), lambda i: (0, i, 0)),
        pl.BlockSpec((1, BN, F), lambda i: (1, i, 0)),
        pl.BlockSpec((F, F), lambda i: (0, 0)),
        pl.BlockSpec((1, F), lambda i: (0, 0)),
    ],
    out_specs=pl.BlockSpec((BN, F), lambda i: (i, 0)),
)


@jax.jit
def kernel(x, edge_index, W, b):
    x2 = x.reshape(N, F)
    rows_r = edge_index[0].reshape(NC, NS, NCH, K)
    cols_r = edge_index[1].reshape(NC, NS, NCH, K)
    rows_c = edge_index[0].reshape(NC, NS, NCHC, KC)
    cols_c = edge_index[1].reshape(NC, NS, NCHC, KC)
    zrow = jnp.zeros((NP,), jnp.float32)
    zagg = jnp.zeros((RPT, F), jnp.float32)

    hist2 = _sc_degree(rows_r, zrow)
    h3 = hist2.reshape(NC, NP, 1)
    y = _tc_scale(h3, h3, x2)
    agg2 = _sc_aggregate(cols_c, rows_c, y, zagg)
    out = _tc_final(h3, h3, y, agg2, agg2, W, b.reshape(1, F))
    return out.reshape(1, N, F)
